# baseline (speedup 1.0000x reference)
import functools

import jax
import jax.numpy as jnp
from jax import lax
from jax.experimental import pallas as pl
from jax.experimental.pallas import tpu as pltpu
from jax.experimental.pallas import tpu_sc as plsc

N = 10000
E = 160000
H = 256
MSG = 64
KEY = 32

NPAD = 10112
ACC_W = 128

BN = 1000
GRID = N // BN


def _proj0_body(feat, h, wft, wht, bp, bih, bhh,
                pf_o, gf_o, v_o, s_o, q_o, gh_o):
    a = jnp.dot(feat[...].astype(jnp.bfloat16), wft[...],
                preferred_element_type=jnp.float32)
    b = jnp.dot(h[...].astype(jnp.bfloat16), wht[...],
                preferred_element_type=jnp.float32)
    pf = a[:, :128] + bp[...]
    gf = a[:, 128:] + bih[...]
    p1 = pf + b[:, :128]
    pf_o[...] = pf
    gf_o[...] = gf
    v_o[...] = p1[:, :MSG].astype(jnp.bfloat16)
    s_o[...] = p1[:, MSG:MSG + KEY].astype(jnp.bfloat16)
    q_o[...] = p1[:, MSG + KEY:].astype(jnp.bfloat16)
    gh_o[...] = b[:, 128:] + bhh[...]


def _gru_core(c, gf, gh, h, wihct):
    gi = gf + jnp.dot(c.astype(jnp.bfloat16), wihct,
                      preferred_element_type=jnp.float32)
    i_r, i_z, i_n = gi[:, :H], gi[:, H:2 * H], gi[:, 2 * H:]
    h_r, h_z, h_n = gh[:, :H], gh[:, H:2 * H], gh[:, 2 * H:]
    r = jax.nn.sigmoid(i_r + h_r)
    z = jax.nn.sigmoid(i_z + h_z)
    n = jnp.tanh(i_n + r * h_n)
    return (1.0 - z) * n + z * h


def _finalize_c(acc):
    a = jnp.sum(acc[...], axis=0)
    den = jnp.sum(a[:, :16], axis=1)
    num = a[:, 16:16 + MSG]
    return num * (1.0 / jnp.maximum(den, 1e-30))[:, None]


def _gru_proj_body(acc, gf, gh, h, pf, wihct, wht, bhh,
                   h1_o, v_o, s_o, q_o, gh_o):
    c = _finalize_c(acc)
    h1 = _gru_core(c, gf[...], gh[...], h[...], wihct[...])
    h1_o[...] = h1
    b2 = jnp.dot(h1.astype(jnp.bfloat16), wht[...],
                 preferred_element_type=jnp.float32)
    p2 = pf[...] + b2[:, :128]
    v_o[...] = p2[:, :MSG].astype(jnp.bfloat16)
    s_o[...] = p2[:, MSG:MSG + KEY].astype(jnp.bfloat16)
    q_o[...] = p2[:, MSG + KEY:].astype(jnp.bfloat16)
    gh_o[...] = b2[:, 128:] + bhh[...]


def _gru_final_body(acc, gf, gh, h, wihct, h2_o):
    c = _finalize_c(acc)
    h2_o[...] = _gru_core(c, gf[...], gh[...], h[...], wihct[...])


def _row_spec(w):
    return pl.BlockSpec((BN, w), lambda i: (i, 0))


def _full_spec(shape):
    return pl.BlockSpec(shape, lambda i: tuple(0 for _ in shape))


def _proj0(feat, h, wft, wht, bp, bih, bhh):
    return pl.pallas_call(
        _proj0_body,
        grid=(GRID,),
        in_specs=[_row_spec(H), _row_spec(H), _full_spec((H, 896)),
                  _full_spec((H, 896)), _full_spec((1, 128)),
                  _full_spec((1, 768)), _full_spec((1, 768))],
        out_specs=[_row_spec(128), _row_spec(768), _row_spec(MSG),
                   _row_spec(KEY), _row_spec(KEY), _row_spec(768)],
        out_shape=[jax.ShapeDtypeStruct((N, 128), jnp.float32),
                   jax.ShapeDtypeStruct((N, 768), jnp.float32),
                   jax.ShapeDtypeStruct((N, MSG), jnp.bfloat16),
                   jax.ShapeDtypeStruct((N, KEY), jnp.bfloat16),
                   jax.ShapeDtypeStruct((N, KEY), jnp.bfloat16),
                   jax.ShapeDtypeStruct((N, 768), jnp.float32)],
        compiler_params=pltpu.CompilerParams(
            dimension_semantics=("parallel",)),
    )(feat, h, wft, wht, bp, bih, bhh)


def _acc_spec(nacc):
    return pl.BlockSpec((nacc, BN, ACC_W), lambda i: (0, i, 0))


def _gru_proj(acc, gf, gh, h, pf, wihct, wht, bhh):
    return pl.pallas_call(
        _gru_proj_body,
        grid=(GRID,),
        in_specs=[_acc_spec(acc.shape[0]),
                  _row_spec(768), _row_spec(768), _row_spec(H),
                  _row_spec(128), _full_spec((MSG, 768)),
                  _full_spec((H, 896)), _full_spec((1, 768))],
        out_specs=[_row_spec(H), _row_spec(MSG), _row_spec(KEY),
                   _row_spec(KEY), _row_spec(768)],
        out_shape=[jax.ShapeDtypeStruct((N, H), jnp.float32),
                   jax.ShapeDtypeStruct((N, MSG), jnp.bfloat16),
                   jax.ShapeDtypeStruct((N, KEY), jnp.bfloat16),
                   jax.ShapeDtypeStruct((N, KEY), jnp.bfloat16),
                   jax.ShapeDtypeStruct((N, 768), jnp.float32)],
        compiler_params=pltpu.CompilerParams(
            dimension_semantics=("parallel",)),
    )(acc, gf, gh, h, pf, wihct, wht, bhh)


def _gru_final(acc, gf, gh, h, wihct):
    return pl.pallas_call(
        _gru_final_body,
        grid=(GRID,),
        in_specs=[_acc_spec(acc.shape[0]),
                  _row_spec(768), _row_spec(768), _row_spec(H),
                  _full_spec((MSG, 768))],
        out_specs=_row_spec(H),
        out_shape=jax.ShapeDtypeStruct((N, H), jnp.float32),
        compiler_params=pltpu.CompilerParams(
            dimension_semantics=("parallel",)),
    )(acc, gf, gh, h, wihct)


NC, NS, L = 2, 16, 16
NW = NC * NS
CHUNK = 128
NCHUNK = 40
EPADT = NW * NCHUNK * CHUNK
RPT = NPAD // NS

_sc_mesh = plsc.VectorSubcoreMesh(core_axis_name="c", subcore_axis_name="s",
                                  num_cores=NC, num_subcores=NS)


def _gather_sqv_body(src_hbm, dst_hbm, s_hbm, q_hbm, v_hbm,
                     g_hbm,
                     src_v, dst_v, s_v, q_v, v_v, gsem, wsem):
    cid = lax.axis_index("c")
    sid = lax.axis_index("s")
    wid = cid * NS + sid
    pltpu.sync_copy(src_hbm.at[wid], src_v)
    pltpu.sync_copy(dst_hbm.at[wid], dst_v)

    def start_gather(t, b):
        pltpu.async_copy(s_hbm.at[src_v.at[t]], s_v.at[b], gsem)
        pltpu.async_copy(q_hbm.at[dst_v.at[t]], q_v.at[b], gsem)
        pltpu.async_copy(v_hbm.at[src_v.at[t]], v_v.at[b], gsem)

    def wait_gather(t, b):
        pltpu.make_async_copy(s_hbm.at[src_v.at[t]], s_v.at[b], gsem).wait()
        pltpu.make_async_copy(q_hbm.at[dst_v.at[t]], q_v.at[b], gsem).wait()
        pltpu.make_async_copy(v_hbm.at[src_v.at[t]], v_v.at[b], gsem).wait()

    def _write_descs(t, b):
        row0 = (wid * NCHUNK + t) * CHUNK
        rows = pl.ds(row0, CHUNK)
        return ((s_v.at[b], g_hbm.at[rows, pl.ds(0, KEY)]),
                (q_v.at[b], g_hbm.at[rows, pl.ds(KEY, KEY)]),
                (v_v.at[b], g_hbm.at[rows, pl.ds(2 * KEY, MSG)]))

    def start_write(t, b):
        for src_r, dst_r in _write_descs(t, b):
            pltpu.async_copy(src_r, dst_r, wsem)

    def wait_write(t, b):
        for src_r, dst_r in _write_descs(t, b):
            pltpu.make_async_copy(src_r, dst_r, wsem).wait()

    start_gather(0, 0)

    def pair_body(tt, carry):
        for b in range(2):
            t = 2 * tt + b
            nb = 1 - b
            wait_gather(t, b)

            @pl.when(t + 1 < NCHUNK)
            def _():
                @pl.when(t >= 1)
                def _():
                    wait_write(t - 1, nb)
                start_gather(t + 1, nb)

            start_write(t, b)
        return carry

    lax.fori_loop(0, NCHUNK // 2, pair_body, 0)
    wait_write(NCHUNK - 2, 0)
    wait_write(NCHUNK - 1, 1)


_gather_sqv = pl.kernel(
    _gather_sqv_body,
    out_type=jax.ShapeDtypeStruct((EPADT, 2 * KEY + MSG), jnp.bfloat16),
    mesh=_sc_mesh,
    compiler_params=pltpu.CompilerParams(use_tc_tiling_on_sc=False),
    scratch_types=[
        pltpu.VMEM((NCHUNK, CHUNK), jnp.int32),
        pltpu.VMEM((NCHUNK, CHUNK), jnp.int32),
        pltpu.VMEM((2, CHUNK, KEY), jnp.bfloat16),
        pltpu.VMEM((2, CHUNK, KEY), jnp.bfloat16),
        pltpu.VMEM((2, CHUNK, MSG), jnp.bfloat16),
        pltpu.SemaphoreType.DMA,
        pltpu.SemaphoreType.DMA,
    ],
)


BE = 2048


def _build_m_body(g, m_o):
    gb = g[...].astype(jnp.float32)
    p = gb[:, 0:KEY] * gb[:, KEY:2 * KEY]
    ee = jnp.exp(jnp.sum(p, axis=1) * (1.0 / KEY))[:, None]
    m_o[...] = jnp.concatenate(
        [ee, jnp.zeros((BE, L - 1), jnp.float32),
         ee * gb[:, 2 * KEY:],
         jnp.zeros((BE, ACC_W - L - MSG), jnp.float32)], axis=1)


def _build_m(g):
    return pl.pallas_call(
        _build_m_body,
        grid=(EPADT // BE,),
        in_specs=[pl.BlockSpec((BE, 2 * KEY + MSG), lambda i: (i, 0))],
        out_specs=pl.BlockSpec((BE, ACC_W), lambda i: (i, 0)),
        out_shape=jax.ShapeDtypeStruct((EPADT, ACC_W), jnp.float32),
        compiler_params=pltpu.CompilerParams(
            dimension_semantics=("parallel",)),
    )(g)


def _scatter_acc_body(dst_hbm, m_hbm, zeros_hbm, out_hbm,
                      dst_v, o_v, lsem, acc_sh):
    cid = lax.axis_index("c")
    sid = lax.axis_index("s")
    wid = cid * NS + sid
    pltpu.sync_copy(zeros_hbm.at[pl.ds(sid * RPT, RPT)],
                    acc_sh.at[pl.ds(sid * RPT, RPT)])
    pltpu.sync_copy(dst_hbm.at[wid], dst_v)
    plsc.subcore_barrier()

    def load_m(t, b):
        row0 = (wid * NCHUNK + t) * CHUNK
        return pltpu.make_async_copy(m_hbm.at[pl.ds(row0, CHUNK)],
                                     o_v.at[b], lsem)

    def start_load(t, b):
        row0 = (wid * NCHUNK + t) * CHUNK
        pltpu.async_copy(m_hbm.at[pl.ds(row0, CHUNK)], o_v.at[b], lsem)

    start_load(0, 0)

    def pair_body(tt, carry):
        for b in range(2):
            t = 2 * tt + b
            load_m(t, b).wait()

            @pl.when(t + 1 < NCHUNK)
            def _():
                start_load(t + 1, 1 - b)

            pltpu.sync_copy(o_v.at[b], acc_sh.at[dst_v.at[t]], add=True)
        return carry

    lax.fori_loop(0, NCHUNK // 2, pair_body, 0)
    plsc.subcore_barrier()
    pltpu.sync_copy(acc_sh.at[pl.ds(sid * RPT, RPT)],
                    out_hbm.at[cid, pl.ds(sid * RPT, RPT)])


_scatter_acc = pl.kernel(
    _scatter_acc_body,
    out_type=jax.ShapeDtypeStruct((NC, NPAD, ACC_W), jnp.float32),
    mesh=_sc_mesh,
    compiler_params=pltpu.CompilerParams(use_tc_tiling_on_sc=False),
    scratch_types=[
        pltpu.VMEM((NCHUNK, CHUNK), jnp.int32),
        pltpu.VMEM((2, CHUNK, ACC_W), jnp.float32),
        pltpu.SemaphoreType.DMA,
        pltpu.VMEM_SHARED((NPAD, ACC_W), jnp.float32),
    ],
)


def _edge_pass(src_pad, dst_gpad, dst_spad, s, q, v, zeros):
    g = _gather_sqv(src_pad, dst_gpad, s, q, v)
    m = _build_m(g)
    return _scatter_acc(dst_spad, m, zeros)


def kernel(feat, h, edge_index, W_val, b_val, W_sign, b_sign, W_que, b_que,
           W_ih, b_ih, W_hh, b_hh):
    src = edge_index[0]
    dst = edge_index[1]
    wf = jnp.concatenate([W_val[:, :H], W_sign[:, :H], W_que[:, :H],
                          W_ih[:, :H]], axis=0)
    wh = jnp.concatenate([W_val[:, H:], W_sign[:, H:], W_que[:, H:],
                          W_hh], axis=0)
    wft = wf.T.astype(jnp.bfloat16)
    wht = wh.T.astype(jnp.bfloat16)
    wihct = W_ih[:, H:].T.astype(jnp.bfloat16)
    bp = jnp.concatenate([b_val, b_sign, b_que])[None]
    bih = b_ih[None]
    bhh = b_hh[None]

    pad = EPADT - E
    src_pad = jnp.concatenate(
        [src, jnp.zeros((pad,), jnp.int32)]).reshape(NW, NCHUNK, CHUNK)
    dst_gpad = jnp.concatenate(
        [dst, jnp.zeros((pad,), jnp.int32)]).reshape(NW, NCHUNK, CHUNK)
    dst_spad = jnp.concatenate(
        [dst, jnp.full((pad,), N, jnp.int32)]).reshape(NW, NCHUNK, CHUNK)
    zeros = jnp.zeros((NPAD, ACC_W), jnp.float32)

    pf, gf, v1, s1, q1, gh1 = _proj0(feat, h, wft, wht, bp, bih, bhh)
    acc1 = _edge_pass(src_pad, dst_gpad, dst_spad, s1, q1, v1, zeros)
    h1, v2, s2, q2, gh2 = _gru_proj(acc1, gf, gh1, h, pf, wihct, wht, bhh)
    acc2 = _edge_pass(src_pad, dst_gpad, dst_spad, s2, q2, v2, zeros)
    h2 = _gru_final(acc2, gf, gh2, h1, wihct)
    return (h2, h2)


# speedup vs baseline: 1.3719x; 1.3719x over previous
import functools

import jax
import jax.numpy as jnp
from jax import lax
from jax.experimental import pallas as pl
from jax.experimental.pallas import tpu as pltpu
from jax.experimental.pallas import tpu_sc as plsc

N = 10000
E = 160000
H = 256
MSG = 64
KEY = 32

NPAD = 10112
ACC_W = 128

BN = 1000
GRID = N // BN


def _bf16_bits(x):
    r = x.astype(jnp.bfloat16).astype(jnp.float32)
    return lax.bitcast_convert_type(r, jnp.uint32)


def _pack_bf16(x):
    w = x.shape[1]
    lo = _bf16_bits(x[:, :w // 2]) >> 16
    hi = _bf16_bits(x[:, w // 2:]) & jnp.uint32(0xFFFF0000)
    return lax.bitcast_convert_type(lo | hi, jnp.float32)


def _unpack_bf16(xp):
    u = lax.bitcast_convert_type(xp, jnp.uint32)
    lo = lax.bitcast_convert_type(u << 16, jnp.float32)
    hi = lax.bitcast_convert_type(u & jnp.uint32(0xFFFF0000), jnp.float32)
    return lo, hi


def _proj0_body(feat, h, wft, wht, bp, bih, bhh,
                pf_o, gf_o, v_o, s_o, q_o, gh_o):
    a = jnp.dot(feat[...].astype(jnp.bfloat16), wft[...],
                preferred_element_type=jnp.float32)
    b = jnp.dot(h[...].astype(jnp.bfloat16), wht[...],
                preferred_element_type=jnp.float32)
    pf = a[:, :128] + bp[...]
    gf = a[:, 128:] + bih[...]
    p1 = pf + b[:, :128]
    pf_o[...] = pf
    gf_o[...] = gf
    v_o[...] = _pack_bf16(p1[:, :MSG])
    s_o[...] = _pack_bf16(p1[:, MSG:MSG + KEY])
    q_o[...] = _pack_bf16(p1[:, MSG + KEY:])
    gh_o[...] = b[:, 128:] + bhh[...]


def _gru_core(c, gf, gh, h, wihct):
    gi = gf + jnp.dot(c.astype(jnp.bfloat16), wihct,
                      preferred_element_type=jnp.float32)
    i_r, i_z, i_n = gi[:, :H], gi[:, H:2 * H], gi[:, 2 * H:]
    h_r, h_z, h_n = gh[:, :H], gh[:, H:2 * H], gh[:, 2 * H:]
    r = jax.nn.sigmoid(i_r + h_r)
    z = jax.nn.sigmoid(i_z + h_z)
    n = jnp.tanh(i_n + r * h_n)
    return (1.0 - z) * n + z * h


def _finalize_c(acc):
    a = jnp.sum(acc[...], axis=0)
    den = jnp.sum(a[:, :16], axis=1)
    num = a[:, 16:16 + MSG]
    return num * (1.0 / jnp.maximum(den, 1e-30))[:, None]


def _gru_proj_body(acc, gf, gh, h, pf, wihct, wht, bhh,
                   h1_o, v_o, s_o, q_o, gh_o):
    c = _finalize_c(acc)
    h1 = _gru_core(c, gf[...], gh[...], h[...], wihct[...])
    h1_o[...] = h1
    b2 = jnp.dot(h1.astype(jnp.bfloat16), wht[...],
                 preferred_element_type=jnp.float32)
    p2 = pf[...] + b2[:, :128]
    v_o[...] = _pack_bf16(p2[:, :MSG])
    s_o[...] = _pack_bf16(p2[:, MSG:MSG + KEY])
    q_o[...] = _pack_bf16(p2[:, MSG + KEY:])
    gh_o[...] = b2[:, 128:] + bhh[...]


def _gru_final_body(acc, gf, gh, h, wihct, h2_o):
    c = _finalize_c(acc)
    h2_o[...] = _gru_core(c, gf[...], gh[...], h[...], wihct[...])


def _row_spec(w):
    return pl.BlockSpec((BN, w), lambda i: (i, 0))


def _full_spec(shape):
    return pl.BlockSpec(shape, lambda i: tuple(0 for _ in shape))


def _proj0(feat, h, wft, wht, bp, bih, bhh):
    return pl.pallas_call(
        _proj0_body,
        grid=(GRID,),
        in_specs=[_row_spec(H), _row_spec(H), _full_spec((H, 896)),
                  _full_spec((H, 896)), _full_spec((1, 128)),
                  _full_spec((1, 768)), _full_spec((1, 768))],
        out_specs=[_row_spec(128), _row_spec(768), _row_spec(MSG // 2),
                   _row_spec(KEY // 2), _row_spec(KEY // 2), _row_spec(768)],
        out_shape=[jax.ShapeDtypeStruct((N, 128), jnp.float32),
                   jax.ShapeDtypeStruct((N, 768), jnp.float32),
                   jax.ShapeDtypeStruct((N, MSG // 2), jnp.float32),
                   jax.ShapeDtypeStruct((N, KEY // 2), jnp.float32),
                   jax.ShapeDtypeStruct((N, KEY // 2), jnp.float32),
                   jax.ShapeDtypeStruct((N, 768), jnp.float32)],
        compiler_params=pltpu.CompilerParams(
            dimension_semantics=("parallel",)),
    )(feat, h, wft, wht, bp, bih, bhh)


def _acc_spec(nacc):
    return pl.BlockSpec((nacc, BN, ACC_W), lambda i: (0, i, 0))


def _gru_proj(acc, gf, gh, h, pf, wihct, wht, bhh):
    return pl.pallas_call(
        _gru_proj_body,
        grid=(GRID,),
        in_specs=[_acc_spec(acc.shape[0]),
                  _row_spec(768), _row_spec(768), _row_spec(H),
                  _row_spec(128), _full_spec((MSG, 768)),
                  _full_spec((H, 896)), _full_spec((1, 768))],
        out_specs=[_row_spec(H), _row_spec(MSG // 2), _row_spec(KEY // 2),
                   _row_spec(KEY // 2), _row_spec(768)],
        out_shape=[jax.ShapeDtypeStruct((N, H), jnp.float32),
                   jax.ShapeDtypeStruct((N, MSG // 2), jnp.float32),
                   jax.ShapeDtypeStruct((N, KEY // 2), jnp.float32),
                   jax.ShapeDtypeStruct((N, KEY // 2), jnp.float32),
                   jax.ShapeDtypeStruct((N, 768), jnp.float32)],
        compiler_params=pltpu.CompilerParams(
            dimension_semantics=("parallel",)),
    )(acc, gf, gh, h, pf, wihct, wht, bhh)


def _gru_final(acc, gf, gh, h, wihct):
    return pl.pallas_call(
        _gru_final_body,
        grid=(GRID,),
        in_specs=[_acc_spec(acc.shape[0]),
                  _row_spec(768), _row_spec(768), _row_spec(H),
                  _full_spec((MSG, 768))],
        out_specs=_row_spec(H),
        out_shape=jax.ShapeDtypeStruct((N, H), jnp.float32),
        compiler_params=pltpu.CompilerParams(
            dimension_semantics=("parallel",)),
    )(acc, gf, gh, h, wihct)


NC, NS, L = 2, 16, 16
NW = NC * NS
CHUNK = 128
NCHUNK = 40
EPADT = NW * NCHUNK * CHUNK
RPT = NPAD // NS

_sc_mesh = plsc.VectorSubcoreMesh(core_axis_name="c", subcore_axis_name="s",
                                  num_cores=NC, num_subcores=NS)


def _gather_sqv_body(src_hbm, dst_hbm, s_hbm, q_hbm, v_hbm,
                     g_hbm,
                     src_v, dst_v, s_v, q_v, v_v, gsem, wsem):
    cid = lax.axis_index("c")
    sid = lax.axis_index("s")
    wid = cid * NS + sid
    pltpu.sync_copy(src_hbm.at[wid], src_v)
    pltpu.sync_copy(dst_hbm.at[wid], dst_v)

    def start_gather(t, b):
        pltpu.async_copy(s_hbm.at[src_v.at[t]], s_v.at[b], gsem)
        pltpu.async_copy(q_hbm.at[dst_v.at[t]], q_v.at[b], gsem)
        pltpu.async_copy(v_hbm.at[src_v.at[t]], v_v.at[b], gsem)

    def wait_gather(t, b):
        pltpu.make_async_copy(s_hbm.at[src_v.at[t]], s_v.at[b], gsem).wait()
        pltpu.make_async_copy(q_hbm.at[dst_v.at[t]], q_v.at[b], gsem).wait()
        pltpu.make_async_copy(v_hbm.at[src_v.at[t]], v_v.at[b], gsem).wait()

    def _write_descs(t, b):
        row0 = (wid * NCHUNK + t) * CHUNK
        rows = pl.ds(row0, CHUNK)
        return ((s_v.at[b], g_hbm.at[rows, pl.ds(0, KEY // 2)]),
                (q_v.at[b], g_hbm.at[rows, pl.ds(KEY // 2, KEY // 2)]),
                (v_v.at[b], g_hbm.at[rows, pl.ds(KEY, MSG // 2)]))

    def start_write(t, b):
        for src_r, dst_r in _write_descs(t, b):
            pltpu.async_copy(src_r, dst_r, wsem)

    def wait_write(t, b):
        for src_r, dst_r in _write_descs(t, b):
            pltpu.make_async_copy(src_r, dst_r, wsem).wait()

    start_gather(0, 0)

    def pair_body(tt, carry):
        for b in range(2):
            t = 2 * tt + b
            nb = 1 - b
            wait_gather(t, b)

            @pl.when(t + 1 < NCHUNK)
            def _():
                @pl.when(t >= 1)
                def _():
                    wait_write(t - 1, nb)
                start_gather(t + 1, nb)

            start_write(t, b)
        return carry

    lax.fori_loop(0, NCHUNK // 2, pair_body, 0)
    wait_write(NCHUNK - 2, 0)
    wait_write(NCHUNK - 1, 1)


_gather_sqv = pl.kernel(
    _gather_sqv_body,
    out_type=jax.ShapeDtypeStruct((EPADT, ACC_W), jnp.float32),
    mesh=_sc_mesh,
    compiler_params=pltpu.CompilerParams(use_tc_tiling_on_sc=False),
    scratch_types=[
        pltpu.VMEM((NCHUNK, CHUNK), jnp.int32),
        pltpu.VMEM((NCHUNK, CHUNK), jnp.int32),
        pltpu.VMEM((2, CHUNK, KEY // 2), jnp.float32),
        pltpu.VMEM((2, CHUNK, KEY // 2), jnp.float32),
        pltpu.VMEM((2, CHUNK, MSG // 2), jnp.float32),
        pltpu.SemaphoreType.DMA,
        pltpu.SemaphoreType.DMA,
    ],
)


BE = 2048


def _build_m_body(g, m_o):
    gb = g[...]
    s_lo, s_hi = _unpack_bf16(gb[:, 0:KEY // 2])
    q_lo, q_hi = _unpack_bf16(gb[:, KEY // 2:KEY])
    v_lo, v_hi = _unpack_bf16(gb[:, KEY:KEY + MSG // 2])
    p = s_lo * q_lo + s_hi * q_hi
    ee = jnp.exp(jnp.sum(p, axis=1) * (1.0 / KEY))[:, None]
    m_o[...] = jnp.concatenate(
        [ee, jnp.zeros((BE, L - 1), jnp.float32),
         ee * v_lo, ee * v_hi,
         jnp.zeros((BE, ACC_W - L - MSG), jnp.float32)], axis=1)


def _build_m(g):
    return pl.pallas_call(
        _build_m_body,
        grid=(EPADT // BE,),
        in_specs=[pl.BlockSpec((BE, ACC_W), lambda i: (i, 0))],
        out_specs=pl.BlockSpec((BE, ACC_W), lambda i: (i, 0)),
        out_shape=jax.ShapeDtypeStruct((EPADT, ACC_W), jnp.float32),
        compiler_params=pltpu.CompilerParams(
            dimension_semantics=("parallel",)),
    )(g)


def _scatter_acc_body(dst_hbm, m_hbm, zeros_hbm, out_hbm,
                      dst_v, o_v, lsem, acc_sh):
    cid = lax.axis_index("c")
    sid = lax.axis_index("s")
    wid = cid * NS + sid
    pltpu.sync_copy(zeros_hbm.at[pl.ds(sid * RPT, RPT)],
                    acc_sh.at[pl.ds(sid * RPT, RPT)])
    pltpu.sync_copy(dst_hbm.at[wid], dst_v)
    plsc.subcore_barrier()

    def load_m(t, b):
        row0 = (wid * NCHUNK + t) * CHUNK
        return pltpu.make_async_copy(m_hbm.at[pl.ds(row0, CHUNK)],
                                     o_v.at[b], lsem)

    def start_load(t, b):
        row0 = (wid * NCHUNK + t) * CHUNK
        pltpu.async_copy(m_hbm.at[pl.ds(row0, CHUNK)], o_v.at[b], lsem)

    start_load(0, 0)

    def pair_body(tt, carry):
        for b in range(2):
            t = 2 * tt + b
            load_m(t, b).wait()

            @pl.when(t + 1 < NCHUNK)
            def _():
                start_load(t + 1, 1 - b)

            pltpu.sync_copy(o_v.at[b], acc_sh.at[dst_v.at[t]], add=True)
        return carry

    lax.fori_loop(0, NCHUNK // 2, pair_body, 0)
    plsc.subcore_barrier()
    pltpu.sync_copy(acc_sh.at[pl.ds(sid * RPT, RPT)],
                    out_hbm.at[cid, pl.ds(sid * RPT, RPT)])


_scatter_acc = pl.kernel(
    _scatter_acc_body,
    out_type=jax.ShapeDtypeStruct((NC, NPAD, ACC_W), jnp.float32),
    mesh=_sc_mesh,
    compiler_params=pltpu.CompilerParams(use_tc_tiling_on_sc=False),
    scratch_types=[
        pltpu.VMEM((NCHUNK, CHUNK), jnp.int32),
        pltpu.VMEM((2, CHUNK, ACC_W), jnp.float32),
        pltpu.SemaphoreType.DMA,
        pltpu.VMEM_SHARED((NPAD, ACC_W), jnp.float32),
    ],
)


def _edge_pass(src_pad, dst_gpad, dst_spad, s, q, v, zeros):
    g = _gather_sqv(src_pad, dst_gpad, s, q, v)
    m = _build_m(g)
    return _scatter_acc(dst_spad, m, zeros)


def kernel(feat, h, edge_index, W_val, b_val, W_sign, b_sign, W_que, b_que,
           W_ih, b_ih, W_hh, b_hh):
    src = edge_index[0]
    dst = edge_index[1]
    wf = jnp.concatenate([W_val[:, :H], W_sign[:, :H], W_que[:, :H],
                          W_ih[:, :H]], axis=0)
    wh = jnp.concatenate([W_val[:, H:], W_sign[:, H:], W_que[:, H:],
                          W_hh], axis=0)
    wft = wf.T.astype(jnp.bfloat16)
    wht = wh.T.astype(jnp.bfloat16)
    wihct = W_ih[:, H:].T.astype(jnp.bfloat16)
    bp = jnp.concatenate([b_val, b_sign, b_que])[None]
    bih = b_ih[None]
    bhh = b_hh[None]

    pad = EPADT - E
    src_pad = jnp.concatenate(
        [src, jnp.zeros((pad,), jnp.int32)]).reshape(NW, NCHUNK, CHUNK)
    dst_gpad = jnp.concatenate(
        [dst, jnp.zeros((pad,), jnp.int32)]).reshape(NW, NCHUNK, CHUNK)
    dst_spad = jnp.concatenate(
        [dst, jnp.full((pad,), N, jnp.int32)]).reshape(NW, NCHUNK, CHUNK)
    zeros = jnp.zeros((NPAD, ACC_W), jnp.float32)

    pf, gf, v1, s1, q1, gh1 = _proj0(feat, h, wft, wht, bp, bih, bhh)
    acc1 = _edge_pass(src_pad, dst_gpad, dst_spad, s1, q1, v1, zeros)
    h1, v2, s2, q2, gh2 = _gru_proj(acc1, gf, gh1, h, pf, wihct, wht, bhh)
    acc2 = _edge_pass(src_pad, dst_gpad, dst_spad, s2, q2, v2, zeros)
    h2 = _gru_final(acc2, gf, gh2, h1, wihct)
    return (h2, h2)


# speedup vs baseline: 1.5724x; 1.1461x over previous
import functools

import jax
import jax.numpy as jnp
from jax import lax
from jax.experimental import pallas as pl
from jax.experimental.pallas import tpu as pltpu
from jax.experimental.pallas import tpu_sc as plsc

N = 10000
E = 160000
H = 256
MSG = 64
KEY = 32

NPAD = 10112
ACC_W = 128

BN = 1000
GRID = N // BN


def _bf16_bits(x):
    r = x.astype(jnp.bfloat16).astype(jnp.float32)
    return lax.bitcast_convert_type(r, jnp.uint32)


def _pack_bf16(x):
    w = x.shape[1]
    lo = _bf16_bits(x[:, :w // 2]) >> 16
    hi = _bf16_bits(x[:, w // 2:]) & jnp.uint32(0xFFFF0000)
    return lax.bitcast_convert_type(lo | hi, jnp.float32)


def _unpack_bf16(xp):
    u = lax.bitcast_convert_type(xp, jnp.uint32)
    lo = lax.bitcast_convert_type(u << 16, jnp.float32)
    hi = lax.bitcast_convert_type(u & jnp.uint32(0xFFFF0000), jnp.float32)
    return lo, hi


def _proj0_body(feat, h, wft, wht, bp, bih, bhh,
                pf_o, gf_o, v_o, s_o, q_o, gh_o):
    a = jnp.dot(feat[...].astype(jnp.bfloat16), wft[...],
                preferred_element_type=jnp.float32)
    b = jnp.dot(h[...].astype(jnp.bfloat16), wht[...],
                preferred_element_type=jnp.float32)
    pf = a[:, :128] + bp[...]
    gf = a[:, 128:] + bih[...]
    p1 = pf + b[:, :128]
    pf_o[...] = pf
    gf_o[...] = gf
    v_o[...] = _pack_bf16(p1[:, :MSG])
    s_o[...] = _pack_bf16(p1[:, MSG:MSG + KEY])
    q_o[...] = _pack_bf16(p1[:, MSG + KEY:])
    gh_o[...] = b[:, 128:] + bhh[...]


def _gru_core(c, gf, gh, h, wihct):
    gi = gf + jnp.dot(c.astype(jnp.bfloat16), wihct,
                      preferred_element_type=jnp.float32)
    i_r, i_z, i_n = gi[:, :H], gi[:, H:2 * H], gi[:, 2 * H:]
    h_r, h_z, h_n = gh[:, :H], gh[:, H:2 * H], gh[:, 2 * H:]
    r = jax.nn.sigmoid(i_r + h_r)
    z = jax.nn.sigmoid(i_z + h_z)
    n = jnp.tanh(i_n + r * h_n)
    return (1.0 - z) * n + z * h


def _finalize_c(acca, accb):
    a = jnp.sum(acca[...], axis=0) + jnp.sum(accb[...], axis=0)
    den = jnp.sum(a[:, :16], axis=1)
    num = a[:, 16:16 + MSG]
    return num * (1.0 / jnp.maximum(den, 1e-30))[:, None]


def _gru_proj_body(acca, accb, gf, gh, h, pf, wihct, wht, bhh,
                   h1_o, v_o, s_o, q_o, gh_o):
    c = _finalize_c(acca, accb)
    h1 = _gru_core(c, gf[...], gh[...], h[...], wihct[...])
    h1_o[...] = h1
    b2 = jnp.dot(h1.astype(jnp.bfloat16), wht[...],
                 preferred_element_type=jnp.float32)
    p2 = pf[...] + b2[:, :128]
    v_o[...] = _pack_bf16(p2[:, :MSG])
    s_o[...] = _pack_bf16(p2[:, MSG:MSG + KEY])
    q_o[...] = _pack_bf16(p2[:, MSG + KEY:])
    gh_o[...] = b2[:, 128:] + bhh[...]


def _gru_final_body(acca, accb, gf, gh, h, wihct, h2_o):
    c = _finalize_c(acca, accb)
    h2_o[...] = _gru_core(c, gf[...], gh[...], h[...], wihct[...])


def _row_spec(w):
    return pl.BlockSpec((BN, w), lambda i: (i, 0))


def _full_spec(shape):
    return pl.BlockSpec(shape, lambda i: tuple(0 for _ in shape))


def _proj0(feat, h, wft, wht, bp, bih, bhh):
    return pl.pallas_call(
        _proj0_body,
        grid=(GRID,),
        in_specs=[_row_spec(H), _row_spec(H), _full_spec((H, 896)),
                  _full_spec((H, 896)), _full_spec((1, 128)),
                  _full_spec((1, 768)), _full_spec((1, 768))],
        out_specs=[_row_spec(128), _row_spec(768), _row_spec(MSG // 2),
                   _row_spec(KEY // 2), _row_spec(KEY // 2), _row_spec(768)],
        out_shape=[jax.ShapeDtypeStruct((N, 128), jnp.float32),
                   jax.ShapeDtypeStruct((N, 768), jnp.float32),
                   jax.ShapeDtypeStruct((N, MSG // 2), jnp.float32),
                   jax.ShapeDtypeStruct((N, KEY // 2), jnp.float32),
                   jax.ShapeDtypeStruct((N, KEY // 2), jnp.float32),
                   jax.ShapeDtypeStruct((N, 768), jnp.float32)],
        compiler_params=pltpu.CompilerParams(
            dimension_semantics=("parallel",)),
    )(feat, h, wft, wht, bp, bih, bhh)


def _acc_spec(nacc):
    return pl.BlockSpec((nacc, BN, ACC_W), lambda i: (0, i, 0))


def _gru_proj(acca, accb, gf, gh, h, pf, wihct, wht, bhh):
    return pl.pallas_call(
        _gru_proj_body,
        grid=(GRID,),
        in_specs=[_acc_spec(acca.shape[0]), _acc_spec(accb.shape[0]),
                  _row_spec(768), _row_spec(768), _row_spec(H),
                  _row_spec(128), _full_spec((MSG, 768)),
                  _full_spec((H, 896)), _full_spec((1, 768))],
        out_specs=[_row_spec(H), _row_spec(MSG // 2), _row_spec(KEY // 2),
                   _row_spec(KEY // 2), _row_spec(768)],
        out_shape=[jax.ShapeDtypeStruct((N, H), jnp.float32),
                   jax.ShapeDtypeStruct((N, MSG // 2), jnp.float32),
                   jax.ShapeDtypeStruct((N, KEY // 2), jnp.float32),
                   jax.ShapeDtypeStruct((N, KEY // 2), jnp.float32),
                   jax.ShapeDtypeStruct((N, 768), jnp.float32)],
        compiler_params=pltpu.CompilerParams(
            dimension_semantics=("parallel",)),
    )(acca, accb, gf, gh, h, pf, wihct, wht, bhh)


def _gru_final(acca, accb, gf, gh, h, wihct):
    return pl.pallas_call(
        _gru_final_body,
        grid=(GRID,),
        in_specs=[_acc_spec(acca.shape[0]), _acc_spec(accb.shape[0]),
                  _row_spec(768), _row_spec(768), _row_spec(H),
                  _full_spec((MSG, 768))],
        out_specs=_row_spec(H),
        out_shape=jax.ShapeDtypeStruct((N, H), jnp.float32),
        compiler_params=pltpu.CompilerParams(
            dimension_semantics=("parallel",)),
    )(acca, accb, gf, gh, h, wihct)


NC, NS, L = 2, 16, 16
NW = NC * NS
CHUNK = 128
NCHUNK = 20
EPADT = NW * NCHUNK * CHUNK
RPT = NPAD // NS

_sc_mesh = plsc.VectorSubcoreMesh(core_axis_name="c", subcore_axis_name="s",
                                  num_cores=NC, num_subcores=NS)


def _gather_sqv_body(src_hbm, dst_hbm, s_hbm, q_hbm, v_hbm,
                     g_hbm,
                     src_v, dst_v, s_v, q_v, v_v, gsem, wsem):
    cid = lax.axis_index("c")
    sid = lax.axis_index("s")
    wid = cid * NS + sid
    pltpu.sync_copy(src_hbm.at[wid], src_v)
    pltpu.sync_copy(dst_hbm.at[wid], dst_v)

    def start_gather(t, b):
        pltpu.async_copy(s_hbm.at[src_v.at[t]], s_v.at[b], gsem)
        pltpu.async_copy(q_hbm.at[dst_v.at[t]], q_v.at[b], gsem)
        pltpu.async_copy(v_hbm.at[src_v.at[t]], v_v.at[b], gsem)

    def wait_gather(t, b):
        pltpu.make_async_copy(s_hbm.at[src_v.at[t]], s_v.at[b], gsem).wait()
        pltpu.make_async_copy(q_hbm.at[dst_v.at[t]], q_v.at[b], gsem).wait()
        pltpu.make_async_copy(v_hbm.at[src_v.at[t]], v_v.at[b], gsem).wait()

    def _write_descs(t, b):
        row0 = (wid * NCHUNK + t) * CHUNK
        rows = pl.ds(row0, CHUNK)
        return ((s_v.at[b], g_hbm.at[rows, pl.ds(0, KEY // 2)]),
                (q_v.at[b], g_hbm.at[rows, pl.ds(KEY // 2, KEY // 2)]),
                (v_v.at[b], g_hbm.at[rows, pl.ds(KEY, MSG // 2)]))

    def start_write(t, b):
        for src_r, dst_r in _write_descs(t, b):
            pltpu.async_copy(src_r, dst_r, wsem)

    def wait_write(t, b):
        for src_r, dst_r in _write_descs(t, b):
            pltpu.make_async_copy(src_r, dst_r, wsem).wait()

    start_gather(0, 0)

    def pair_body(tt, carry):
        for b in range(2):
            t = 2 * tt + b
            nb = 1 - b
            wait_gather(t, b)

            @pl.when(t + 1 < NCHUNK)
            def _():
                @pl.when(t >= 1)
                def _():
                    wait_write(t - 1, nb)
                start_gather(t + 1, nb)

            start_write(t, b)
        return carry

    lax.fori_loop(0, NCHUNK // 2, pair_body, 0)
    wait_write(NCHUNK - 2, 0)
    wait_write(NCHUNK - 1, 1)


_gather_sqv = pl.kernel(
    _gather_sqv_body,
    out_type=jax.ShapeDtypeStruct((EPADT, ACC_W), jnp.float32),
    mesh=_sc_mesh,
    compiler_params=pltpu.CompilerParams(use_tc_tiling_on_sc=False),
    scratch_types=[
        pltpu.VMEM((NCHUNK, CHUNK), jnp.int32),
        pltpu.VMEM((NCHUNK, CHUNK), jnp.int32),
        pltpu.VMEM((2, CHUNK, KEY // 2), jnp.float32),
        pltpu.VMEM((2, CHUNK, KEY // 2), jnp.float32),
        pltpu.VMEM((2, CHUNK, MSG // 2), jnp.float32),
        pltpu.SemaphoreType.DMA,
        pltpu.SemaphoreType.DMA,
    ],
)


BE = 2048


def _build_m_body(g, m_o):
    gb = g[...]
    s_lo, s_hi = _unpack_bf16(gb[:, 0:KEY // 2])
    q_lo, q_hi = _unpack_bf16(gb[:, KEY // 2:KEY])
    v_lo, v_hi = _unpack_bf16(gb[:, KEY:KEY + MSG // 2])
    p = s_lo * q_lo + s_hi * q_hi
    ee = jnp.exp(jnp.sum(p, axis=1) * (1.0 / KEY))[:, None]
    m_o[...] = jnp.concatenate(
        [ee, jnp.zeros((BE, L - 1), jnp.float32),
         ee * v_lo, ee * v_hi,
         jnp.zeros((BE, ACC_W - L - MSG), jnp.float32)], axis=1)


def _build_m(g):
    return pl.pallas_call(
        _build_m_body,
        grid=(EPADT // BE,),
        in_specs=[pl.BlockSpec((BE, ACC_W), lambda i: (i, 0))],
        out_specs=pl.BlockSpec((BE, ACC_W), lambda i: (i, 0)),
        out_shape=jax.ShapeDtypeStruct((EPADT, ACC_W), jnp.float32),
        compiler_params=pltpu.CompilerParams(
            dimension_semantics=("parallel",)),
    )(g)


def _scatter_acc_body(dst_hbm, m_hbm, zeros_hbm, out_hbm,
                      dst_v, o_v, lsem, acc_sh):
    cid = lax.axis_index("c")
    sid = lax.axis_index("s")
    wid = cid * NS + sid
    pltpu.sync_copy(zeros_hbm.at[pl.ds(sid * RPT, RPT)],
                    acc_sh.at[pl.ds(sid * RPT, RPT)])
    pltpu.sync_copy(dst_hbm.at[wid], dst_v)
    plsc.subcore_barrier()

    def load_m(t, b):
        row0 = (wid * NCHUNK + t) * CHUNK
        return pltpu.make_async_copy(m_hbm.at[pl.ds(row0, CHUNK)],
                                     o_v.at[b], lsem)

    def start_load(t, b):
        row0 = (wid * NCHUNK + t) * CHUNK
        pltpu.async_copy(m_hbm.at[pl.ds(row0, CHUNK)], o_v.at[b], lsem)

    start_load(0, 0)

    def pair_body(tt, carry):
        for b in range(2):
            t = 2 * tt + b
            load_m(t, b).wait()

            @pl.when(t + 1 < NCHUNK)
            def _():
                start_load(t + 1, 1 - b)

            pltpu.sync_copy(o_v.at[b], acc_sh.at[dst_v.at[t]], add=True)
        return carry

    lax.fori_loop(0, NCHUNK // 2, pair_body, 0)
    plsc.subcore_barrier()
    pltpu.sync_copy(acc_sh.at[pl.ds(sid * RPT, RPT)],
                    out_hbm.at[cid, pl.ds(sid * RPT, RPT)])


_scatter_acc = pl.kernel(
    _scatter_acc_body,
    out_type=jax.ShapeDtypeStruct((NC, NPAD, ACC_W), jnp.float32),
    mesh=_sc_mesh,
    compiler_params=pltpu.CompilerParams(use_tc_tiling_on_sc=False),
    scratch_types=[
        pltpu.VMEM((NCHUNK, CHUNK), jnp.int32),
        pltpu.VMEM((2, CHUNK, ACC_W), jnp.float32),
        pltpu.SemaphoreType.DMA,
        pltpu.VMEM_SHARED((NPAD, ACC_W), jnp.float32),
    ],
)


def _edge_pass(halves, s, q, v, zeros):
    accs = []
    for src_h, dst_gh, dst_sh in halves:
        g = _gather_sqv(src_h, dst_gh, s, q, v)
        m = _build_m(g)
        accs.append(_scatter_acc(dst_sh, m, zeros))
    return accs


def kernel(feat, h, edge_index, W_val, b_val, W_sign, b_sign, W_que, b_que,
           W_ih, b_ih, W_hh, b_hh):
    src = edge_index[0]
    dst = edge_index[1]
    wf = jnp.concatenate([W_val[:, :H], W_sign[:, :H], W_que[:, :H],
                          W_ih[:, :H]], axis=0)
    wh = jnp.concatenate([W_val[:, H:], W_sign[:, H:], W_que[:, H:],
                          W_hh], axis=0)
    wft = wf.T.astype(jnp.bfloat16)
    wht = wh.T.astype(jnp.bfloat16)
    wihct = W_ih[:, H:].T.astype(jnp.bfloat16)
    bp = jnp.concatenate([b_val, b_sign, b_que])[None]
    bih = b_ih[None]
    bhh = b_hh[None]

    pad = 2 * EPADT - E
    src_pad = jnp.concatenate(
        [src, jnp.zeros((pad,), jnp.int32)]).reshape(NW, 2 * NCHUNK, CHUNK)
    dst_gpad = jnp.concatenate(
        [dst, jnp.zeros((pad,), jnp.int32)]).reshape(NW, 2 * NCHUNK, CHUNK)
    dst_spad = jnp.concatenate(
        [dst, jnp.full((pad,), N, jnp.int32)]).reshape(NW, 2 * NCHUNK, CHUNK)
    halves = [(src_pad[:, :NCHUNK], dst_gpad[:, :NCHUNK],
               dst_spad[:, :NCHUNK]),
              (src_pad[:, NCHUNK:], dst_gpad[:, NCHUNK:],
               dst_spad[:, NCHUNK:])]
    zeros = jnp.zeros((NPAD, ACC_W), jnp.float32)

    pf, gf, v1, s1, q1, gh1 = _proj0(feat, h, wft, wht, bp, bih, bhh)
    acc1a, acc1b = _edge_pass(halves, s1, q1, v1, zeros)
    h1, v2, s2, q2, gh2 = _gru_proj(acc1a, acc1b, gf, gh1, h, pf,
                                    wihct, wht, bhh)
    acc2a, acc2b = _edge_pass(halves, s2, q2, v2, zeros)
    h2 = _gru_final(acc2a, acc2b, gf, gh2, h1, wihct)
    return (h2, h2)
